# baseline (speedup 1.0000x reference)
import functools

import jax
import jax.numpy as jnp
from jax import lax
from jax.experimental import pallas as pl
from jax.experimental.pallas import tpu as pltpu
from jax.experimental.pallas import tpu_sc as plsc

B = 16384
L = 20
NNEG = 5
D = 64
PD = 128
NC, NS, LANES = 2, 16, 16
NW = NC * NS
EPW = B // NW
CB = 32
NBLK = EPW // CB
TN = 1 + NNEG
SLOTS = 16
PAD_SCORE = 1e4
TBLK = 1024
P = 499712
NPB = 489


def _tc_pack_rows(table):
    t = jnp.transpose(table)

    def body(x1_ref, x2_ref, o_ref):
        o_ref[:, 0:D] = jnp.transpose(x1_ref[...])
        o_ref[:, D:PD] = jnp.transpose(x2_ref[...])

    return pl.pallas_call(
        body,
        grid=(NPB,),
        in_specs=[
            pl.BlockSpec((D, TBLK),
                         lambda j: (0, jnp.where(j == NPB - 1, 976, j))),
            pl.BlockSpec((D, TBLK), lambda j: (0, NPB - 1 + j)),
        ],
        out_specs=pl.BlockSpec((TBLK, PD), lambda j: (j, 0)),
        out_shape=jax.ShapeDtypeStruct((NPB * TBLK, PD), jnp.float32),
        compiler_params=pltpu.CompilerParams(
            dimension_semantics=("arbitrary",)),
    )(t, t)


def _sc_scores(ctx_flat, ctx32_flat, tn_flat, syn0p, syn1p):
    mesh = plsc.VectorSubcoreMesh(core_axis_name="c", subcore_axis_name="s")

    @functools.partial(
        pl.kernel,
        out_type=jax.ShapeDtypeStruct((B * SLOTS,), jnp.float32),
        mesh=mesh,
        compiler_params=pltpu.CompilerParams(
            needs_layout_passes=False, use_tc_tiling_on_sc=True),
        scratch_types=[
            pltpu.VMEM((CB * L,), jnp.int32),
            pltpu.VMEM((CB * L,), jnp.int32),
            pltpu.VMEM((CB * 2 * LANES,), jnp.int32),
            pltpu.VMEM((CB * TN + LANES,), jnp.int32),
            pltpu.VMEM((CB * TN,), jnp.int32),
            pltpu.VMEM((CB * L, PD), jnp.float32),
            pltpu.VMEM((CB * TN, PD), jnp.float32),
            pltpu.VMEM((CB * SLOTS,), jnp.float32),
            pltpu.SemaphoreType.DMA,
        ],
    )
    def k(ctx_hbm, ctx32_hbm, tn_hbm, syn0_hbm, syn1_hbm, out_hbm,
          idx_ctx, idxp_ctx, idx32, idx_tn, idxp_tn, rows_ctx, rows_tn,
          scores, sem):
        wid = lax.axis_index("s") * NC + lax.axis_index("c")
        lane = lax.iota(jnp.int32, LANES)

        def block(g, carry):
            base = wid * EPW + g * CB
            pltpu.sync_copy(ctx_hbm.at[pl.ds(base * L, CB * L)], idx_ctx)
            pltpu.sync_copy(ctx32_hbm.at[pl.ds(base * 2 * LANES, CB * 2 * LANES)], idx32)
            pltpu.sync_copy(tn_hbm.at[pl.ds(base * TN, CB * TN)],
                            idx_tn.at[pl.ds(0, CB * TN)])
            for t in range(CB * L // LANES):
                v = idx_ctx[pl.ds(t * LANES, LANES)]
                idxp_ctx[pl.ds(t * LANES, LANES)] = jnp.where(v >= P, v - P, v)
            for t in range(CB * TN // LANES):
                v = idx_tn[pl.ds(t * LANES, LANES)]
                idxp_tn[pl.ds(t * LANES, LANES)] = jnp.where(v >= P, v - P, v)
            handles = []
            for t in range(CB * L // 128):
                handles.append(pltpu.async_copy(
                    syn0_hbm.at[idxp_ctx.at[pl.ds(t * 128, 128)]],
                    rows_ctx.at[pl.ds(t * 128, 128)], sem))
            for t in range(2):
                half = CB * TN // 2
                handles.append(pltpu.async_copy(
                    syn1_hbm.at[idxp_tn.at[pl.ds(t * half, half)]],
                    rows_tn.at[pl.ds(t * half, half)], sem))
            for h in handles:
                h.wait()

            def elem(e, carry2):
                v1 = idx32[pl.ds(e * 2 * LANES, LANES)]
                v2 = idx32[pl.ds(e * 2 * LANES + LANES, LANES)]
                cnt = (jnp.sum(jnp.where(v1 != 0, 1.0, 0.0))
                       + jnp.sum(jnp.where(v2 != 0, 1.0, 0.0)))
                rcp = 1.0 / jnp.full((LANES,), cnt, jnp.float32)
                iv1 = idx_ctx[pl.ds(e * L, LANES)]
                iv2 = idx_ctx[pl.ds(e * L + (L - LANES), LANES)]
                pv1 = jnp.where((iv1 >= P) & (iv1 < 2 * P), D, 0)
                pv2 = jnp.where((iv2 >= P) & (iv2 < 2 * P), D, 0)
                offs = [pv1[r] for r in range(LANES)]
                offs += [pv2[r] for r in range(2 * LANES - L, LANES)]
                mean = [None] * 4
                for r in range(L):
                    for c in range(4):
                        v = rows_ctx[e * L + r, pl.ds(offs[r] + c * LANES, LANES)]
                        mean[c] = v if r == 0 else mean[c] + v
                mean = [m * rcp for m in mean]
                tvi = idx_tn[pl.ds(e * TN, LANES)]
                tv = jnp.where((tvi >= P) & (tvi < 2 * P), D, 0)
                s = jnp.full((LANES,), PAD_SCORE, jnp.float32)
                for n in range(TN):
                    off = tv[n]
                    acc = mean[0] * rows_tn[e * TN + n, pl.ds(off, LANES)]
                    for c in range(1, 4):
                        acc = acc + mean[c] * rows_tn[
                            e * TN + n, pl.ds(off + c * LANES, LANES)]
                    val = jnp.sum(acc) if n == 0 else -jnp.sum(acc)
                    s = jnp.where(lane == n, val, s)
                scores[pl.ds(e * SLOTS, SLOTS)] = s
                return carry2

            lax.fori_loop(0, CB, elem, 0)
            pltpu.sync_copy(scores, out_hbm.at[pl.ds(base * SLOTS, CB * SLOTS)])
            return carry

        lax.fori_loop(0, NBLK, block, 0)

    return k(ctx_flat, ctx32_flat, tn_flat, syn0p, syn1p)


def _tc_loss(scores2d):
    def body(s_ref, o_ref):
        x = s_ref[...]
        ls = jnp.minimum(x, 0.0) - jnp.log1p(jnp.exp(-jnp.abs(x)))
        o_ref[...] = jnp.full((1, 1), -jnp.sum(ls), jnp.float32)

    out = pl.pallas_call(
        body,
        out_shape=jax.ShapeDtypeStruct((1, 1), jnp.float32),
    )(scores2d)
    return out[0, 0]


def kernel(target, context, negatives, syn0, syn1):
    ctx_flat = context.reshape(-1).astype(jnp.int32)
    ctx32 = jnp.pad(context.astype(jnp.int32), ((0, 0), (0, 2 * LANES - L)))
    tn = jnp.concatenate([target[:, None].astype(jnp.int32),
                          negatives.astype(jnp.int32)], axis=1)
    syn0p = _tc_pack_rows(syn0)
    syn1p = _tc_pack_rows(syn1)
    scores = _sc_scores(ctx_flat, ctx32.reshape(-1), tn.reshape(-1),
                        syn0p, syn1p)
    return _tc_loss(scores.reshape(B * SLOTS // 128, 128))


# speedup vs baseline: 1.2710x; 1.2710x over previous
import functools

import jax
import jax.numpy as jnp
from jax import lax
from jax.experimental import pallas as pl
from jax.experimental.pallas import tpu as pltpu
from jax.experimental.pallas import tpu_sc as plsc

B = 16384
L = 20
NNEG = 5
D = 64
PD = 128
NC, NS, LANES = 2, 16, 16
NW = NC * NS
EPW = B // NW
CB = 32
NBLK = EPW // CB
TN = 1 + NNEG
SLOTS = 16
PAD_SCORE = 1e4
TBLK = 2048
P = 499712
NPB = 245
PLAST = 488


def _tc_pack_rows(table):
    t = jnp.transpose(table)

    def body(x1_ref, x2_ref, o_ref):
        eye = (lax.broadcasted_iota(jnp.int32, (D, D), 0)
               == lax.broadcasted_iota(jnp.int32, (D, D), 1)
               ).astype(jnp.float32)
        dn = (((0,), (0,)), ((), ()))
        o_ref[:, 0:D] = lax.dot_general(
            x1_ref[...], eye, dn, preferred_element_type=jnp.float32)
        o_ref[:, D:PD] = lax.dot_general(
            x2_ref[...], eye, dn, preferred_element_type=jnp.float32)

    return pl.pallas_call(
        body,
        grid=(NPB,),
        in_specs=[
            pl.BlockSpec((D, TBLK),
                         lambda j: (0, jnp.where(j == NPB - 1, PLAST, j))),
            pl.BlockSpec((D, TBLK), lambda j: (0, NPB - 1 + j)),
        ],
        out_specs=pl.BlockSpec((TBLK, PD), lambda j: (j, 0)),
        out_shape=jax.ShapeDtypeStruct((NPB * TBLK, PD), jnp.float32),
        compiler_params=pltpu.CompilerParams(
            dimension_semantics=("arbitrary",),
            fuse_transposed_lhs_in_matmul=True),
    )(t, t)


def _sc_scores(ctx_flat, ctx32_flat, tn_flat, syn0p, syn1p):
    mesh = plsc.VectorSubcoreMesh(core_axis_name="c", subcore_axis_name="s")

    @functools.partial(
        pl.kernel,
        out_type=jax.ShapeDtypeStruct((B * SLOTS,), jnp.float32),
        mesh=mesh,
        compiler_params=pltpu.CompilerParams(
            needs_layout_passes=False, use_tc_tiling_on_sc=True),
        scratch_types=[
            pltpu.VMEM((CB * L,), jnp.int32),
            pltpu.VMEM((CB * L,), jnp.int32),
            pltpu.VMEM((CB * 2 * LANES,), jnp.int32),
            pltpu.VMEM((CB * TN + LANES,), jnp.int32),
            pltpu.VMEM((CB * TN,), jnp.int32),
            pltpu.VMEM((CB * L, PD), jnp.float32),
            pltpu.VMEM((CB * TN, PD), jnp.float32),
            pltpu.VMEM((CB * SLOTS,), jnp.float32),
            pltpu.SemaphoreType.DMA,
        ],
    )
    def k(ctx_hbm, ctx32_hbm, tn_hbm, syn0_hbm, syn1_hbm, out_hbm,
          idx_ctx, idxp_ctx, idx32, idx_tn, idxp_tn, rows_ctx, rows_tn,
          scores, sem):
        wid = lax.axis_index("s") * NC + lax.axis_index("c")
        lane = lax.iota(jnp.int32, LANES)

        def block(g, carry):
            base = wid * EPW + g * CB
            pltpu.sync_copy(ctx_hbm.at[pl.ds(base * L, CB * L)], idx_ctx)
            pltpu.sync_copy(ctx32_hbm.at[pl.ds(base * 2 * LANES, CB * 2 * LANES)], idx32)
            pltpu.sync_copy(tn_hbm.at[pl.ds(base * TN, CB * TN)],
                            idx_tn.at[pl.ds(0, CB * TN)])
            for t in range(CB * L // LANES):
                v = idx_ctx[pl.ds(t * LANES, LANES)]
                idxp_ctx[pl.ds(t * LANES, LANES)] = jnp.where(v >= P, v - P, v)
            for t in range(CB * TN // LANES):
                v = idx_tn[pl.ds(t * LANES, LANES)]
                idxp_tn[pl.ds(t * LANES, LANES)] = jnp.where(v >= P, v - P, v)
            handles = []
            for t in range(CB * L // 128):
                handles.append(pltpu.async_copy(
                    syn0_hbm.at[idxp_ctx.at[pl.ds(t * 128, 128)]],
                    rows_ctx.at[pl.ds(t * 128, 128)], sem))
            for t in range(2):
                half = CB * TN // 2
                handles.append(pltpu.async_copy(
                    syn1_hbm.at[idxp_tn.at[pl.ds(t * half, half)]],
                    rows_tn.at[pl.ds(t * half, half)], sem))
            for h in handles:
                h.wait()

            def elem(e, carry2):
                v1 = idx32[pl.ds(e * 2 * LANES, LANES)]
                v2 = idx32[pl.ds(e * 2 * LANES + LANES, LANES)]
                cnt = (jnp.sum(jnp.where(v1 != 0, 1.0, 0.0))
                       + jnp.sum(jnp.where(v2 != 0, 1.0, 0.0)))
                rcp = 1.0 / jnp.full((LANES,), cnt, jnp.float32)
                iv1 = idx_ctx[pl.ds(e * L, LANES)]
                iv2 = idx_ctx[pl.ds(e * L + (L - LANES), LANES)]
                pv1 = jnp.where((iv1 >= P) & (iv1 < 2 * P), D, 0)
                pv2 = jnp.where((iv2 >= P) & (iv2 < 2 * P), D, 0)
                offs = [pv1[r] for r in range(LANES)]
                offs += [pv2[r] for r in range(2 * LANES - L, LANES)]
                mean = [None] * 4
                for r in range(L):
                    for c in range(4):
                        v = rows_ctx[e * L + r, pl.ds(offs[r] + c * LANES, LANES)]
                        mean[c] = v if r == 0 else mean[c] + v
                mean = [m * rcp for m in mean]
                tvi = idx_tn[pl.ds(e * TN, LANES)]
                tv = jnp.where((tvi >= P) & (tvi < 2 * P), D, 0)
                s = jnp.full((LANES,), PAD_SCORE, jnp.float32)
                for n in range(TN):
                    off = tv[n]
                    acc = mean[0] * rows_tn[e * TN + n, pl.ds(off, LANES)]
                    for c in range(1, 4):
                        acc = acc + mean[c] * rows_tn[
                            e * TN + n, pl.ds(off + c * LANES, LANES)]
                    val = jnp.sum(acc) if n == 0 else -jnp.sum(acc)
                    s = jnp.where(lane == n, val, s)
                scores[pl.ds(e * SLOTS, SLOTS)] = s
                return carry2

            lax.fori_loop(0, CB, elem, 0)
            pltpu.sync_copy(scores, out_hbm.at[pl.ds(base * SLOTS, CB * SLOTS)])
            return carry

        lax.fori_loop(0, NBLK, block, 0)

    return k(ctx_flat, ctx32_flat, tn_flat, syn0p, syn1p)


def _tc_loss(scores2d):
    def body(s_ref, o_ref):
        x = s_ref[...]
        ls = jnp.minimum(x, 0.0) - jnp.log1p(jnp.exp(-jnp.abs(x)))
        o_ref[...] = jnp.full((1, 1), -jnp.sum(ls), jnp.float32)

    out = pl.pallas_call(
        body,
        out_shape=jax.ShapeDtypeStruct((1, 1), jnp.float32),
    )(scores2d)
    return out[0, 0]


def kernel(target, context, negatives, syn0, syn1):
    ctx_flat = context.reshape(-1).astype(jnp.int32)
    ctx32 = jnp.pad(context.astype(jnp.int32), ((0, 0), (0, 2 * LANES - L)))
    tn = jnp.concatenate([target[:, None].astype(jnp.int32),
                          negatives.astype(jnp.int32)], axis=1)
    syn0p = _tc_pack_rows(syn0)
    syn1p = _tc_pack_rows(syn1)
    scores = _sc_scores(ctx_flat, ctx32.reshape(-1), tn.reshape(-1),
                        syn0p, syn1p)
    return _tc_loss(scores.reshape(B * SLOTS // 128, 128))


# speedup vs baseline: 1.6932x; 1.3322x over previous
import functools

import jax
import jax.numpy as jnp
from jax import lax
from jax.experimental import pallas as pl
from jax.experimental.pallas import tpu as pltpu
from jax.experimental.pallas import tpu_sc as plsc

B = 16384
L = 20
NNEG = 5
D = 64
PD = 128
NC, NS, LANES = 2, 16, 16
NW = NC * NS
EPW = B // NW
CB = 32
NBLK = EPW // CB
TN = 1 + NNEG
SLOTS = 16
PAD_SCORE = 1e4
TBLK = 4096
P = 499712
NPB = 123
PLAST = 244


def _tc_pack_rows(table):
    t = jnp.transpose(table)

    def body(x1_ref, x2_ref, o_ref):
        eye = (lax.broadcasted_iota(jnp.int32, (D, D), 0)
               == lax.broadcasted_iota(jnp.int32, (D, D), 1)
               ).astype(jnp.float32)
        dn = (((0,), (0,)), ((), ()))
        o_ref[:, 0:D] = lax.dot_general(
            x1_ref[...], eye, dn, preferred_element_type=jnp.float32)
        o_ref[:, D:PD] = lax.dot_general(
            x2_ref[...], eye, dn, preferred_element_type=jnp.float32)

    return pl.pallas_call(
        body,
        grid=(NPB,),
        in_specs=[
            pl.BlockSpec((D, TBLK),
                         lambda j: (0, jnp.where(j == NPB - 1, PLAST, j))),
            pl.BlockSpec((D, TBLK), lambda j: (0, P // TBLK + j)),
        ],
        out_specs=pl.BlockSpec((TBLK, PD), lambda j: (j, 0)),
        out_shape=jax.ShapeDtypeStruct((NPB * TBLK, PD), jnp.float32),
        compiler_params=pltpu.CompilerParams(
            dimension_semantics=("arbitrary",),
            fuse_transposed_lhs_in_matmul=True),
    )(t, t)


_SC_PARAMS = pltpu.CompilerParams(
    needs_layout_passes=False, use_tc_tiling_on_sc=True)
_MESH = dict(core_axis_name="c", subcore_axis_name="s")


def _sc_pool(ctx_flat, ctx32_flat, syn0p):
    @functools.partial(
        pl.kernel,
        out_type=jax.ShapeDtypeStruct((B * D,), jnp.float32),
        mesh=plsc.VectorSubcoreMesh(**_MESH),
        compiler_params=_SC_PARAMS,
        scratch_types=[
            pltpu.VMEM((CB * L,), jnp.int32),
            pltpu.VMEM((CB * L,), jnp.int32),
            pltpu.VMEM((CB * 2 * LANES,), jnp.int32),
            pltpu.VMEM((CB * L, PD), jnp.float32),
            pltpu.VMEM((CB * D,), jnp.float32),
            pltpu.SemaphoreType.DMA,
        ],
    )
    def k(ctx_hbm, ctx32_hbm, syn0_hbm, out_hbm,
          idx_ctx, idxp_ctx, idx32, rows_ctx, meanb, sem):
        wid = lax.axis_index("s") * NC + lax.axis_index("c")

        def block(g, carry):
            base = wid * EPW + g * CB
            pltpu.sync_copy(ctx_hbm.at[pl.ds(base * L, CB * L)], idx_ctx)
            pltpu.sync_copy(
                ctx32_hbm.at[pl.ds(base * 2 * LANES, CB * 2 * LANES)], idx32)
            for t in range(CB * L // LANES):
                v = idx_ctx[pl.ds(t * LANES, LANES)]
                idxp_ctx[pl.ds(t * LANES, LANES)] = jnp.where(v >= P, v - P, v)
            handles = []
            for t in range(CB * L // 128):
                handles.append(pltpu.async_copy(
                    syn0_hbm.at[idxp_ctx.at[pl.ds(t * 128, 128)]],
                    rows_ctx.at[pl.ds(t * 128, 128)], sem))
            for h in handles:
                h.wait()

            def elem(e, carry2):
                v1 = idx32[pl.ds(e * 2 * LANES, LANES)]
                v2 = idx32[pl.ds(e * 2 * LANES + LANES, LANES)]
                cnt = (jnp.sum(jnp.where(v1 != 0, 1.0, 0.0))
                       + jnp.sum(jnp.where(v2 != 0, 1.0, 0.0)))
                rcp = 1.0 / jnp.full((LANES,), cnt, jnp.float32)
                iv1 = idx_ctx[pl.ds(e * L, LANES)]
                iv2 = idx_ctx[pl.ds(e * L + (L - LANES), LANES)]
                pv1 = jnp.where((iv1 >= P) & (iv1 < 2 * P), D, 0)
                pv2 = jnp.where((iv2 >= P) & (iv2 < 2 * P), D, 0)
                offs = [pv1[r] for r in range(LANES)]
                offs += [pv2[r] for r in range(2 * LANES - L, LANES)]
                for c in range(4):
                    acc = rows_ctx[e * L, pl.ds(offs[0] + c * LANES, LANES)]
                    for r in range(1, L):
                        acc = acc + rows_ctx[
                            e * L + r, pl.ds(offs[r] + c * LANES, LANES)]
                    meanb[pl.ds(e * D + c * LANES, LANES)] = acc * rcp
                return carry2

            lax.fori_loop(0, CB, elem, 0)
            pltpu.sync_copy(meanb, out_hbm.at[pl.ds(base * D, CB * D)])
            return carry

        lax.fori_loop(0, NBLK, block, 0)

    return k(ctx_flat, ctx32_flat, syn0p)


def _sc_dots(tn_flat, mean_flat, syn1p):
    @functools.partial(
        pl.kernel,
        out_type=jax.ShapeDtypeStruct((B * SLOTS,), jnp.float32),
        mesh=plsc.VectorSubcoreMesh(**_MESH),
        compiler_params=_SC_PARAMS,
        scratch_types=[
            pltpu.VMEM((CB * TN + LANES,), jnp.int32),
            pltpu.VMEM((CB * TN,), jnp.int32),
            pltpu.VMEM((CB * TN, PD), jnp.float32),
            pltpu.VMEM((CB * D,), jnp.float32),
            pltpu.VMEM((CB * SLOTS,), jnp.float32),
            pltpu.SemaphoreType.DMA,
        ],
    )
    def k(tn_hbm, mean_hbm, syn1_hbm, out_hbm,
          idx_tn, idxp_tn, rows_tn, meanb, scores, sem):
        wid = lax.axis_index("s") * NC + lax.axis_index("c")
        lane = lax.iota(jnp.int32, LANES)

        def block(g, carry):
            base = wid * EPW + g * CB
            pltpu.sync_copy(tn_hbm.at[pl.ds(base * TN, CB * TN)],
                            idx_tn.at[pl.ds(0, CB * TN)])
            pltpu.sync_copy(mean_hbm.at[pl.ds(base * D, CB * D)], meanb)
            for t in range(CB * TN // LANES):
                v = idx_tn[pl.ds(t * LANES, LANES)]
                idxp_tn[pl.ds(t * LANES, LANES)] = jnp.where(v >= P, v - P, v)
            handles = []
            for t in range(2):
                half = CB * TN // 2
                handles.append(pltpu.async_copy(
                    syn1_hbm.at[idxp_tn.at[pl.ds(t * half, half)]],
                    rows_tn.at[pl.ds(t * half, half)], sem))
            for h in handles:
                h.wait()

            def elem(e, carry2):
                mean = [meanb[pl.ds(e * D + c * LANES, LANES)]
                        for c in range(4)]
                tvi = idx_tn[pl.ds(e * TN, LANES)]
                tv = jnp.where((tvi >= P) & (tvi < 2 * P), D, 0)
                s = jnp.full((LANES,), PAD_SCORE, jnp.float32)
                for n in range(TN):
                    off = tv[n]
                    acc = mean[0] * rows_tn[e * TN + n, pl.ds(off, LANES)]
                    for c in range(1, 4):
                        acc = acc + mean[c] * rows_tn[
                            e * TN + n, pl.ds(off + c * LANES, LANES)]
                    val = jnp.sum(acc) if n == 0 else -jnp.sum(acc)
                    s = jnp.where(lane == n, val, s)
                scores[pl.ds(e * SLOTS, SLOTS)] = s
                return carry2

            lax.fori_loop(0, CB, elem, 0)
            pltpu.sync_copy(scores, out_hbm.at[pl.ds(base * SLOTS, CB * SLOTS)])
            return carry

        lax.fori_loop(0, NBLK, block, 0)

    return k(tn_flat, mean_flat, syn1p)


def _tc_loss(scores2d):
    def body(s_ref, o_ref):
        x = s_ref[...]
        ls = jnp.minimum(x, 0.0) - jnp.log1p(jnp.exp(-jnp.abs(x)))
        o_ref[...] = jnp.full((1, 1), -jnp.sum(ls), jnp.float32)

    out = pl.pallas_call(
        body,
        out_shape=jax.ShapeDtypeStruct((1, 1), jnp.float32),
    )(scores2d)
    return out[0, 0]


def kernel(target, context, negatives, syn0, syn1):
    ctx_flat = context.reshape(-1).astype(jnp.int32)
    ctx32 = jnp.pad(context.astype(jnp.int32), ((0, 0), (0, 2 * LANES - L)))
    tn = jnp.concatenate([target[:, None].astype(jnp.int32),
                          negatives.astype(jnp.int32)], axis=1)
    syn0p = _tc_pack_rows(syn0)
    means = _sc_pool(ctx_flat, ctx32.reshape(-1), syn0p)
    syn1p = _tc_pack_rows(syn1)
    scores = _sc_dots(tn.reshape(-1), means, syn1p)
    return _tc_loss(scores.reshape(B * SLOTS // 128, 128))


# speedup vs baseline: 1.8629x; 1.1002x over previous
import functools

import jax
import jax.numpy as jnp
from jax import lax
from jax.experimental import pallas as pl
from jax.experimental.pallas import tpu as pltpu
from jax.experimental.pallas import tpu_sc as plsc

B = 16384
L = 20
NNEG = 5
D = 64
PD = 128
NC, NS, LANES = 2, 16, 16
NW = NC * NS
EPW = B // NW
CB = 32
NBLK = EPW // CB
TN = 1 + NNEG
SLOTS = 16
PAD_SCORE = 1e4
TBLK = 8192
P = 499712
NPB = 62
PLAST = 122


def _tc_pack_rows(table):
    t = jnp.transpose(table)

    def body(x1_ref, x2_ref, o_ref):
        eye = (lax.broadcasted_iota(jnp.int32, (D, D), 0)
               == lax.broadcasted_iota(jnp.int32, (D, D), 1)
               ).astype(jnp.float32)
        dn = (((0,), (0,)), ((), ()))
        o_ref[:, 0:D] = lax.dot_general(
            x1_ref[...], eye, dn, preferred_element_type=jnp.float32)
        o_ref[:, D:PD] = lax.dot_general(
            x2_ref[...], eye, dn, preferred_element_type=jnp.float32)

    return pl.pallas_call(
        body,
        grid=(NPB,),
        in_specs=[
            pl.BlockSpec((D, TBLK),
                         lambda j: (0, jnp.where(j == NPB - 1, PLAST, j))),
            pl.BlockSpec((D, TBLK), lambda j: (0, P // TBLK + j)),
        ],
        out_specs=pl.BlockSpec((TBLK, PD), lambda j: (j, 0)),
        out_shape=jax.ShapeDtypeStruct((NPB * TBLK, PD), jnp.float32),
        compiler_params=pltpu.CompilerParams(
            dimension_semantics=("arbitrary",),
            fuse_transposed_lhs_in_matmul=True),
    )(t, t)


_SC_PARAMS = pltpu.CompilerParams(
    needs_layout_passes=False, use_tc_tiling_on_sc=True)
_MESH = dict(core_axis_name="c", subcore_axis_name="s")


def _sc_pool(ctx_flat, ctx32_flat, syn0p):
    @functools.partial(
        pl.kernel,
        out_type=jax.ShapeDtypeStruct((B * D,), jnp.float32),
        mesh=plsc.VectorSubcoreMesh(**_MESH),
        compiler_params=_SC_PARAMS,
        scratch_types=[
            pltpu.VMEM((CB * L,), jnp.int32),
            pltpu.VMEM((CB * L,), jnp.int32),
            pltpu.VMEM((CB * 2 * LANES,), jnp.int32),
            pltpu.VMEM((CB * L, PD), jnp.float32),
            pltpu.VMEM((CB * D,), jnp.float32),
            pltpu.SemaphoreType.DMA,
        ],
    )
    def k(ctx_hbm, ctx32_hbm, syn0_hbm, out_hbm,
          idx_ctx, idxp_ctx, idx32, rows_ctx, meanb, sem):
        wid = lax.axis_index("s") * NC + lax.axis_index("c")

        def block(g, carry):
            base = wid * EPW + g * CB
            pltpu.sync_copy(ctx_hbm.at[pl.ds(base * L, CB * L)], idx_ctx)
            pltpu.sync_copy(
                ctx32_hbm.at[pl.ds(base * 2 * LANES, CB * 2 * LANES)], idx32)
            for t in range(CB * L // LANES):
                v = idx_ctx[pl.ds(t * LANES, LANES)]
                idxp_ctx[pl.ds(t * LANES, LANES)] = jnp.where(v >= P, v - P, v)
            handles = []
            for t in range(CB * L // 128):
                handles.append(pltpu.async_copy(
                    syn0_hbm.at[idxp_ctx.at[pl.ds(t * 128, 128)]],
                    rows_ctx.at[pl.ds(t * 128, 128)], sem))
            for h in handles:
                h.wait()

            def elem(e, carry2):
                v1 = idx32[pl.ds(e * 2 * LANES, LANES)]
                v2 = idx32[pl.ds(e * 2 * LANES + LANES, LANES)]
                cnt = (jnp.sum(jnp.where(v1 != 0, 1.0, 0.0))
                       + jnp.sum(jnp.where(v2 != 0, 1.0, 0.0)))
                rcp = 1.0 / jnp.full((LANES,), cnt, jnp.float32)
                iv1 = idx_ctx[pl.ds(e * L, LANES)]
                iv2 = idx_ctx[pl.ds(e * L + (L - LANES), LANES)]
                pv1 = jnp.where((iv1 >= P) & (iv1 < 2 * P), D, 0)
                pv2 = jnp.where((iv2 >= P) & (iv2 < 2 * P), D, 0)
                offs = [pv1[r] for r in range(LANES)]
                offs += [pv2[r] for r in range(2 * LANES - L, LANES)]
                for c in range(4):
                    acc = rows_ctx[e * L, pl.ds(offs[0] + c * LANES, LANES)]
                    for r in range(1, L):
                        acc = acc + rows_ctx[
                            e * L + r, pl.ds(offs[r] + c * LANES, LANES)]
                    meanb[pl.ds(e * D + c * LANES, LANES)] = acc * rcp
                return carry2

            lax.fori_loop(0, CB, elem, 0)
            pltpu.sync_copy(meanb, out_hbm.at[pl.ds(base * D, CB * D)])
            return carry

        lax.fori_loop(0, NBLK, block, 0)

    return k(ctx_flat, ctx32_flat, syn0p)


def _sc_dots(tn_flat, mean_flat, syn1p):
    @functools.partial(
        pl.kernel,
        out_type=jax.ShapeDtypeStruct((B * SLOTS,), jnp.float32),
        mesh=plsc.VectorSubcoreMesh(**_MESH),
        compiler_params=_SC_PARAMS,
        scratch_types=[
            pltpu.VMEM((CB * TN + LANES,), jnp.int32),
            pltpu.VMEM((CB * TN,), jnp.int32),
            pltpu.VMEM((CB * TN, PD), jnp.float32),
            pltpu.VMEM((CB * D,), jnp.float32),
            pltpu.VMEM((CB * SLOTS,), jnp.float32),
            pltpu.SemaphoreType.DMA,
        ],
    )
    def k(tn_hbm, mean_hbm, syn1_hbm, out_hbm,
          idx_tn, idxp_tn, rows_tn, meanb, scores, sem):
        wid = lax.axis_index("s") * NC + lax.axis_index("c")
        lane = lax.iota(jnp.int32, LANES)

        def block(g, carry):
            base = wid * EPW + g * CB
            pltpu.sync_copy(tn_hbm.at[pl.ds(base * TN, CB * TN)],
                            idx_tn.at[pl.ds(0, CB * TN)])
            pltpu.sync_copy(mean_hbm.at[pl.ds(base * D, CB * D)], meanb)
            for t in range(CB * TN // LANES):
                v = idx_tn[pl.ds(t * LANES, LANES)]
                idxp_tn[pl.ds(t * LANES, LANES)] = jnp.where(v >= P, v - P, v)
            handles = []
            for t in range(2):
                half = CB * TN // 2
                handles.append(pltpu.async_copy(
                    syn1_hbm.at[idxp_tn.at[pl.ds(t * half, half)]],
                    rows_tn.at[pl.ds(t * half, half)], sem))
            for h in handles:
                h.wait()

            def elem(e, carry2):
                mean = [meanb[pl.ds(e * D + c * LANES, LANES)]
                        for c in range(4)]
                tvi = idx_tn[pl.ds(e * TN, LANES)]
                tv = jnp.where((tvi >= P) & (tvi < 2 * P), D, 0)
                s = jnp.full((LANES,), PAD_SCORE, jnp.float32)
                for n in range(TN):
                    off = tv[n]
                    acc = mean[0] * rows_tn[e * TN + n, pl.ds(off, LANES)]
                    for c in range(1, 4):
                        acc = acc + mean[c] * rows_tn[
                            e * TN + n, pl.ds(off + c * LANES, LANES)]
                    val = jnp.sum(acc) if n == 0 else -jnp.sum(acc)
                    s = jnp.where(lane == n, val, s)
                scores[pl.ds(e * SLOTS, SLOTS)] = s
                return carry2

            lax.fori_loop(0, CB, elem, 0)
            pltpu.sync_copy(scores, out_hbm.at[pl.ds(base * SLOTS, CB * SLOTS)])
            return carry

        lax.fori_loop(0, NBLK, block, 0)

    return k(tn_flat, mean_flat, syn1p)


def _tc_loss(scores2d):
    def body(s_ref, o_ref):
        x = s_ref[...]
        ls = jnp.minimum(x, 0.0) - jnp.log1p(jnp.exp(-jnp.abs(x)))
        o_ref[...] = jnp.full((1, 1), -jnp.sum(ls), jnp.float32)

    out = pl.pallas_call(
        body,
        out_shape=jax.ShapeDtypeStruct((1, 1), jnp.float32),
    )(scores2d)
    return out[0, 0]


def kernel(target, context, negatives, syn0, syn1):
    ctx_flat = context.reshape(-1).astype(jnp.int32)
    ctx32 = jnp.pad(context.astype(jnp.int32), ((0, 0), (0, 2 * LANES - L)))
    tn = jnp.concatenate([target[:, None].astype(jnp.int32),
                          negatives.astype(jnp.int32)], axis=1)
    syn0p = _tc_pack_rows(syn0)
    means = _sc_pool(ctx_flat, ctx32.reshape(-1), syn0p)
    syn1p = _tc_pack_rows(syn1)
    scores = _sc_dots(tn.reshape(-1), means, syn1p)
    return _tc_loss(scores.reshape(B * SLOTS // 128, 128))


# speedup vs baseline: 1.9467x; 1.0450x over previous
import functools

import jax
import jax.numpy as jnp
from jax import lax
from jax.experimental import pallas as pl
from jax.experimental.pallas import tpu as pltpu
from jax.experimental.pallas import tpu_sc as plsc

B = 16384
L = 20
NNEG = 5
D = 64
PD = 128
NC, NS, LANES = 2, 16, 16
NW = NC * NS
EPW = B // NW
CB = 32
NBLK = EPW // CB
TN = 1 + NNEG
SLOTS = 16
PAD_SCORE = 1e4
TBLK = 16384
P = 491520
NPB = 32
PTB = P // TBLK


def _tc_pack_rows(table):
    t = jnp.transpose(table)

    def body(x1_ref, x2_ref, o_ref):
        eye = (lax.broadcasted_iota(jnp.int32, (D, D), 0)
               == lax.broadcasted_iota(jnp.int32, (D, D), 1)
               ).astype(jnp.float32)
        dn = (((0,), (0,)), ((), ()))
        o_ref[:, 0:D] = lax.dot_general(
            x1_ref[...], eye, dn, preferred_element_type=jnp.float32)
        o_ref[:, D:PD] = lax.dot_general(
            x2_ref[...], eye, dn, preferred_element_type=jnp.float32)

    return pl.pallas_call(
        body,
        grid=(NPB,),
        in_specs=[
            pl.BlockSpec((D, TBLK),
                         lambda j: (0, jnp.where(j >= PTB, j + PTB, j))),
            pl.BlockSpec((D, TBLK), lambda j: (0, PTB + j)),
        ],
        out_specs=pl.BlockSpec((TBLK, PD), lambda j: (j, 0)),
        out_shape=jax.ShapeDtypeStruct((NPB * TBLK, PD), jnp.float32),
        compiler_params=pltpu.CompilerParams(
            dimension_semantics=("arbitrary",),
            fuse_transposed_lhs_in_matmul=True),
    )(t, t)


_SC_PARAMS = pltpu.CompilerParams(
    needs_layout_passes=False, use_tc_tiling_on_sc=True)
_MESH = dict(core_axis_name="c", subcore_axis_name="s")


def _sc_pool(ctx_flat, syn0p):
    @functools.partial(
        pl.kernel,
        out_type=jax.ShapeDtypeStruct((B * D,), jnp.float32),
        mesh=plsc.VectorSubcoreMesh(**_MESH),
        compiler_params=_SC_PARAMS,
        scratch_types=[
            pltpu.VMEM((CB * L,), jnp.int32),
            pltpu.VMEM((CB * L,), jnp.int32),
            pltpu.VMEM((CB * L, PD), jnp.float32),
            pltpu.VMEM((CB * D,), jnp.float32),
            pltpu.SemaphoreType.DMA,
        ],
    )
    def k(ctx_hbm, syn0_hbm, out_hbm,
          idx_ctx, idxp_ctx, rows_ctx, meanb, sem):
        wid = lax.axis_index("s") * NC + lax.axis_index("c")
        lane = lax.iota(jnp.int32, LANES)

        def block(g, carry):
            base = wid * EPW + g * CB
            pltpu.sync_copy(ctx_hbm.at[pl.ds(base * L, CB * L)], idx_ctx)
            for t in range(CB * L // LANES):
                v = idx_ctx[pl.ds(t * LANES, LANES)]
                idxp_ctx[pl.ds(t * LANES, LANES)] = jnp.where(v >= P, v - P, v)
            handles = []
            for t in range(CB * L // 128):
                handles.append(pltpu.async_copy(
                    syn0_hbm.at[idxp_ctx.at[pl.ds(t * 128, 128)]],
                    rows_ctx.at[pl.ds(t * 128, 128)], sem))
            for h in handles:
                h.wait()

            def elem(e, carry2):
                iv1 = idx_ctx[pl.ds(e * L, LANES)]
                iv2 = idx_ctx[pl.ds(e * L + (L - LANES), LANES)]
                cnt = (jnp.sum(jnp.where(iv1 != 0, 1.0, 0.0))
                       + jnp.sum(jnp.where((iv2 != 0) & (lane >= 2 * LANES - L),
                                           1.0, 0.0)))
                rcp = 1.0 / jnp.full((LANES,), cnt, jnp.float32)
                pv1 = jnp.where((iv1 >= P) & (iv1 < 2 * P), D, 0)
                pv2 = jnp.where((iv2 >= P) & (iv2 < 2 * P), D, 0)
                offs = [pv1[r] for r in range(LANES)]
                offs += [pv2[r] for r in range(2 * LANES - L, LANES)]
                for c in range(4):
                    acc = rows_ctx[e * L, pl.ds(offs[0] + c * LANES, LANES)]
                    for r in range(1, L):
                        acc = acc + rows_ctx[
                            e * L + r, pl.ds(offs[r] + c * LANES, LANES)]
                    meanb[pl.ds(e * D + c * LANES, LANES)] = acc * rcp
                return carry2

            lax.fori_loop(0, CB, elem, 0)
            pltpu.sync_copy(meanb, out_hbm.at[pl.ds(base * D, CB * D)])
            return carry

        lax.fori_loop(0, NBLK, block, 0)

    return k(ctx_flat, syn0p)


def _sc_dots(tn_flat, mean_flat, syn1p):
    @functools.partial(
        pl.kernel,
        out_type=jax.ShapeDtypeStruct((B * SLOTS,), jnp.float32),
        mesh=plsc.VectorSubcoreMesh(**_MESH),
        compiler_params=_SC_PARAMS,
        scratch_types=[
            pltpu.VMEM((CB * TN + LANES,), jnp.int32),
            pltpu.VMEM((CB * TN,), jnp.int32),
            pltpu.VMEM((CB * TN, PD), jnp.float32),
            pltpu.VMEM((CB * D,), jnp.float32),
            pltpu.VMEM((CB * SLOTS,), jnp.float32),
            pltpu.SemaphoreType.DMA,
        ],
    )
    def k(tn_hbm, mean_hbm, syn1_hbm, out_hbm,
          idx_tn, idxp_tn, rows_tn, meanb, scores, sem):
        wid = lax.axis_index("s") * NC + lax.axis_index("c")
        lane = lax.iota(jnp.int32, LANES)

        def block(g, carry):
            base = wid * EPW + g * CB
            pltpu.sync_copy(tn_hbm.at[pl.ds(base * TN, CB * TN)],
                            idx_tn.at[pl.ds(0, CB * TN)])
            pltpu.sync_copy(mean_hbm.at[pl.ds(base * D, CB * D)], meanb)
            for t in range(CB * TN // LANES):
                v = idx_tn[pl.ds(t * LANES, LANES)]
                idxp_tn[pl.ds(t * LANES, LANES)] = jnp.where(v >= P, v - P, v)
            handles = []
            for t in range(2):
                half = CB * TN // 2
                handles.append(pltpu.async_copy(
                    syn1_hbm.at[idxp_tn.at[pl.ds(t * half, half)]],
                    rows_tn.at[pl.ds(t * half, half)], sem))
            for h in handles:
                h.wait()

            def elem(e, carry2):
                mean = [meanb[pl.ds(e * D + c * LANES, LANES)]
                        for c in range(4)]
                tvi = idx_tn[pl.ds(e * TN, LANES)]
                tv = jnp.where((tvi >= P) & (tvi < 2 * P), D, 0)
                s = jnp.full((LANES,), PAD_SCORE, jnp.float32)
                for n in range(TN):
                    off = tv[n]
                    acc = mean[0] * rows_tn[e * TN + n, pl.ds(off, LANES)]
                    for c in range(1, 4):
                        acc = acc + mean[c] * rows_tn[
                            e * TN + n, pl.ds(off + c * LANES, LANES)]
                    val = jnp.sum(acc) if n == 0 else -jnp.sum(acc)
                    s = jnp.where(lane == n, val, s)
                scores[pl.ds(e * SLOTS, SLOTS)] = s
                return carry2

            lax.fori_loop(0, CB, elem, 0)
            pltpu.sync_copy(scores, out_hbm.at[pl.ds(base * SLOTS, CB * SLOTS)])
            return carry

        lax.fori_loop(0, NBLK, block, 0)

    return k(tn_flat, mean_flat, syn1p)


def _tc_loss(scores2d):
    def body(s_ref, o_ref):
        x = s_ref[...]
        ls = jnp.minimum(x, 0.0) - jnp.log1p(jnp.exp(-jnp.abs(x)))
        o_ref[...] = jnp.full((1, 1), -jnp.sum(ls), jnp.float32)

    out = pl.pallas_call(
        body,
        out_shape=jax.ShapeDtypeStruct((1, 1), jnp.float32),
    )(scores2d)
    return out[0, 0]


def kernel(target, context, negatives, syn0, syn1):
    ctx_flat = context.reshape(-1).astype(jnp.int32)
    tn = jnp.concatenate([target[:, None].astype(jnp.int32),
                          negatives.astype(jnp.int32)], axis=1)
    syn0p = _tc_pack_rows(syn0)
    means = _sc_pool(ctx_flat, syn0p)
    syn1p = _tc_pack_rows(syn1)
    scores = _sc_dots(tn.reshape(-1), means, syn1p)
    return _tc_loss(scores.reshape(B * SLOTS // 128, 128))


# speedup vs baseline: 1.9499x; 1.0017x over previous
import functools

import jax
import jax.numpy as jnp
from jax import lax
from jax.experimental import pallas as pl
from jax.experimental.pallas import tpu as pltpu
from jax.experimental.pallas import tpu_sc as plsc

B = 16384
L = 20
NNEG = 5
D = 64
PD = 128
NC, NS, LANES = 2, 16, 16
NW = NC * NS
EPW = B // NW
CB = 32
NBLK = EPW // CB
TN = 1 + NNEG
SLOTS = 16
PAD_SCORE = 1e4
TBLK = 16384
P = 491520
NPB = 32
PTB = P // TBLK


def _tc_pack_rows(table):
    t = jnp.transpose(table)

    def body(x1_ref, x2_ref, o_ref):
        eye = (lax.broadcasted_iota(jnp.int32, (D, D), 0)
               == lax.broadcasted_iota(jnp.int32, (D, D), 1)
               ).astype(jnp.float32)
        dn = (((0,), (0,)), ((), ()))
        y1 = lax.dot_general(
            x1_ref[...], eye, dn, preferred_element_type=jnp.float32)
        y2 = lax.dot_general(
            x2_ref[...], eye, dn, preferred_element_type=jnp.float32)
        o_ref[...] = jnp.concatenate([y1, y2], axis=1)

    return pl.pallas_call(
        body,
        grid=(NPB,),
        in_specs=[
            pl.BlockSpec((D, TBLK),
                         lambda j: (0, jnp.where(j >= PTB, j + PTB, j))),
            pl.BlockSpec((D, TBLK), lambda j: (0, PTB + j)),
        ],
        out_specs=pl.BlockSpec((TBLK, PD), lambda j: (j, 0)),
        out_shape=jax.ShapeDtypeStruct((NPB * TBLK, PD), jnp.float32),
        compiler_params=pltpu.CompilerParams(
            dimension_semantics=("arbitrary",),
            fuse_transposed_lhs_in_matmul=True),
    )(t, t)


_SC_PARAMS = pltpu.CompilerParams(
    needs_layout_passes=False, use_tc_tiling_on_sc=True)
_MESH = dict(core_axis_name="c", subcore_axis_name="s")


def _sc_pool(ctx_flat, syn0p):
    @functools.partial(
        pl.kernel,
        out_type=jax.ShapeDtypeStruct((B * D,), jnp.float32),
        mesh=plsc.VectorSubcoreMesh(**_MESH),
        compiler_params=_SC_PARAMS,
        scratch_types=[
            pltpu.VMEM((CB * L,), jnp.int32),
            pltpu.VMEM((CB * L,), jnp.int32),
            pltpu.VMEM((CB * L, PD), jnp.float32),
            pltpu.VMEM((CB * D,), jnp.float32),
            pltpu.SemaphoreType.DMA,
        ],
    )
    def k(ctx_hbm, syn0_hbm, out_hbm,
          idx_ctx, idxp_ctx, rows_ctx, meanb, sem):
        wid = lax.axis_index("s") * NC + lax.axis_index("c")
        lane = lax.iota(jnp.int32, LANES)

        def block(g, carry):
            base = wid * EPW + g * CB
            pltpu.sync_copy(ctx_hbm.at[pl.ds(base * L, CB * L)], idx_ctx)
            for t in range(CB * L // LANES):
                v = idx_ctx[pl.ds(t * LANES, LANES)]
                idxp_ctx[pl.ds(t * LANES, LANES)] = jnp.where(v >= P, v - P, v)
            handles = []
            for t in range(CB * L // 128):
                handles.append(pltpu.async_copy(
                    syn0_hbm.at[idxp_ctx.at[pl.ds(t * 128, 128)]],
                    rows_ctx.at[pl.ds(t * 128, 128)], sem))
            for h in handles:
                h.wait()

            def elem(e, carry2):
                iv1 = idx_ctx[pl.ds(e * L, LANES)]
                iv2 = idx_ctx[pl.ds(e * L + (L - LANES), LANES)]
                cnt = (jnp.sum(jnp.where(iv1 != 0, 1.0, 0.0))
                       + jnp.sum(jnp.where((iv2 != 0) & (lane >= 2 * LANES - L),
                                           1.0, 0.0)))
                rcp = 1.0 / jnp.full((LANES,), cnt, jnp.float32)
                pv1 = jnp.where((iv1 >= P) & (iv1 < 2 * P), D, 0)
                pv2 = jnp.where((iv2 >= P) & (iv2 < 2 * P), D, 0)
                offs = [pv1[r] for r in range(LANES)]
                offs += [pv2[r] for r in range(2 * LANES - L, LANES)]
                for c in range(4):
                    acc = rows_ctx[e * L, pl.ds(offs[0] + c * LANES, LANES)]
                    for r in range(1, L):
                        acc = acc + rows_ctx[
                            e * L + r, pl.ds(offs[r] + c * LANES, LANES)]
                    meanb[pl.ds(e * D + c * LANES, LANES)] = acc * rcp
                return carry2

            lax.fori_loop(0, CB, elem, 0)
            pltpu.sync_copy(meanb, out_hbm.at[pl.ds(base * D, CB * D)])
            return carry

        lax.fori_loop(0, NBLK, block, 0)

    return k(ctx_flat, syn0p)


def _sc_dots(tn_flat, mean_flat, syn1p):
    @functools.partial(
        pl.kernel,
        out_type=jax.ShapeDtypeStruct((B * SLOTS,), jnp.float32),
        mesh=plsc.VectorSubcoreMesh(**_MESH),
        compiler_params=_SC_PARAMS,
        scratch_types=[
            pltpu.VMEM((CB * TN + LANES,), jnp.int32),
            pltpu.VMEM((CB * TN,), jnp.int32),
            pltpu.VMEM((CB * TN, PD), jnp.float32),
            pltpu.VMEM((CB * D,), jnp.float32),
            pltpu.VMEM((CB * SLOTS,), jnp.float32),
            pltpu.SemaphoreType.DMA,
        ],
    )
    def k(tn_hbm, mean_hbm, syn1_hbm, out_hbm,
          idx_tn, idxp_tn, rows_tn, meanb, scores, sem):
        wid = lax.axis_index("s") * NC + lax.axis_index("c")
        lane = lax.iota(jnp.int32, LANES)

        def block(g, carry):
            base = wid * EPW + g * CB
            pltpu.sync_copy(tn_hbm.at[pl.ds(base * TN, CB * TN)],
                            idx_tn.at[pl.ds(0, CB * TN)])
            pltpu.sync_copy(mean_hbm.at[pl.ds(base * D, CB * D)], meanb)
            for t in range(CB * TN // LANES):
                v = idx_tn[pl.ds(t * LANES, LANES)]
                idxp_tn[pl.ds(t * LANES, LANES)] = jnp.where(v >= P, v - P, v)
            handles = []
            for t in range(2):
                half = CB * TN // 2
                handles.append(pltpu.async_copy(
                    syn1_hbm.at[idxp_tn.at[pl.ds(t * half, half)]],
                    rows_tn.at[pl.ds(t * half, half)], sem))
            for h in handles:
                h.wait()

            def elem(e, carry2):
                mean = [meanb[pl.ds(e * D + c * LANES, LANES)]
                        for c in range(4)]
                tvi = idx_tn[pl.ds(e * TN, LANES)]
                tv = jnp.where((tvi >= P) & (tvi < 2 * P), D, 0)
                s = jnp.full((LANES,), PAD_SCORE, jnp.float32)
                for n in range(TN):
                    off = tv[n]
                    acc = mean[0] * rows_tn[e * TN + n, pl.ds(off, LANES)]
                    for c in range(1, 4):
                        acc = acc + mean[c] * rows_tn[
                            e * TN + n, pl.ds(off + c * LANES, LANES)]
                    val = jnp.sum(acc) if n == 0 else -jnp.sum(acc)
                    s = jnp.where(lane == n, val, s)
                scores[pl.ds(e * SLOTS, SLOTS)] = s
                return carry2

            lax.fori_loop(0, CB, elem, 0)
            pltpu.sync_copy(scores, out_hbm.at[pl.ds(base * SLOTS, CB * SLOTS)])
            return carry

        lax.fori_loop(0, NBLK, block, 0)

    return k(tn_flat, mean_flat, syn1p)


def _tc_loss(scores2d):
    def body(s_ref, o_ref):
        x = s_ref[...]
        ls = jnp.minimum(x, 0.0) - jnp.log1p(jnp.exp(-jnp.abs(x)))
        o_ref[...] = jnp.full((1, 1), -jnp.sum(ls), jnp.float32)

    out = pl.pallas_call(
        body,
        out_shape=jax.ShapeDtypeStruct((1, 1), jnp.float32),
    )(scores2d)
    return out[0, 0]


def kernel(target, context, negatives, syn0, syn1):
    ctx_flat = context.reshape(-1).astype(jnp.int32)
    tn = jnp.concatenate([target[:, None].astype(jnp.int32),
                          negatives.astype(jnp.int32)], axis=1)
    syn0p = _tc_pack_rows(syn0)
    means = _sc_pool(ctx_flat, syn0p)
    syn1p = _tc_pack_rows(syn1)
    scores = _sc_dots(tn.reshape(-1), means, syn1p)
    return _tc_loss(scores.reshape(B * SLOTS // 128, 128))


# speedup vs baseline: 2.0112x; 1.0314x over previous
import functools

import jax
import jax.numpy as jnp
from jax import lax
from jax.experimental import pallas as pl
from jax.experimental.pallas import tpu as pltpu
from jax.experimental.pallas import tpu_sc as plsc

B = 16384
L = 20
NNEG = 5
D = 64
PD = 128
NC, NS, LANES = 2, 16, 16
NW = NC * NS
EPW = B // NW
CB = 32
NBLK = EPW // CB
TN = 1 + NNEG
SLOTS = 16
PAD_SCORE = 1e4
CB2 = 64
TBLK = 16384
P = 491520
NPB = 32
PTB = P // TBLK


def _tc_pack_rows(table):
    t = jnp.transpose(table)

    def body(x1_ref, x2_ref, o_ref):
        eye = (lax.broadcasted_iota(jnp.int32, (D, D), 0)
               == lax.broadcasted_iota(jnp.int32, (D, D), 1)
               ).astype(jnp.float32)
        dn = (((0,), (0,)), ((), ()))
        y1 = lax.dot_general(
            x1_ref[...], eye, dn, preferred_element_type=jnp.float32)
        y2 = lax.dot_general(
            x2_ref[...], eye, dn, preferred_element_type=jnp.float32)
        o_ref[...] = jnp.concatenate([y1, y2], axis=1)

    return pl.pallas_call(
        body,
        grid=(NPB,),
        in_specs=[
            pl.BlockSpec((D, TBLK),
                         lambda j: (0, jnp.where(j >= PTB, j + PTB, j))),
            pl.BlockSpec((D, TBLK), lambda j: (0, PTB + j)),
        ],
        out_specs=pl.BlockSpec((TBLK, PD), lambda j: (j, 0)),
        out_shape=jax.ShapeDtypeStruct((NPB * TBLK, PD), jnp.float32),
        compiler_params=pltpu.CompilerParams(
            dimension_semantics=("arbitrary",),
            fuse_transposed_lhs_in_matmul=True),
    )(t, t)


_SC_PARAMS = pltpu.CompilerParams(
    needs_layout_passes=False, use_tc_tiling_on_sc=True)
_MESH = dict(core_axis_name="c", subcore_axis_name="s")


def _sc_pool(ctx_flat, syn0p):
    @functools.partial(
        pl.kernel,
        out_type=jax.ShapeDtypeStruct((B * D,), jnp.float32),
        mesh=plsc.VectorSubcoreMesh(**_MESH),
        compiler_params=_SC_PARAMS,
        scratch_types=[
            pltpu.VMEM((CB * L,), jnp.int32),
            pltpu.VMEM((CB * L,), jnp.int32),
            pltpu.VMEM((CB * L, PD), jnp.float32),
            pltpu.VMEM((CB * D,), jnp.float32),
            pltpu.SemaphoreType.DMA,
        ],
    )
    def k(ctx_hbm, syn0_hbm, out_hbm,
          idx_ctx, idxp_ctx, rows_ctx, meanb, sem):
        wid = lax.axis_index("s") * NC + lax.axis_index("c")
        lane = lax.iota(jnp.int32, LANES)

        def block(g, carry):
            base = wid * EPW + g * CB
            pltpu.sync_copy(ctx_hbm.at[pl.ds(base * L, CB * L)], idx_ctx)
            for t in range(CB * L // LANES):
                v = idx_ctx[pl.ds(t * LANES, LANES)]
                idxp_ctx[pl.ds(t * LANES, LANES)] = jnp.where(v >= P, v - P, v)
            handles = []
            for t in range(CB * L // 128):
                handles.append(pltpu.async_copy(
                    syn0_hbm.at[idxp_ctx.at[pl.ds(t * 128, 128)]],
                    rows_ctx.at[pl.ds(t * 128, 128)], sem))
            for h in handles:
                h.wait()

            def elem(e, carry2):
                iv1 = idx_ctx[pl.ds(e * L, LANES)]
                iv2 = idx_ctx[pl.ds(e * L + (L - LANES), LANES)]
                cnt = (jnp.sum(jnp.where(iv1 != 0, 1.0, 0.0))
                       + jnp.sum(jnp.where((iv2 != 0) & (lane >= 2 * LANES - L),
                                           1.0, 0.0)))
                rcp = 1.0 / jnp.full((LANES,), cnt, jnp.float32)
                pv1 = jnp.where((iv1 >= P) & (iv1 < 2 * P), D, 0)
                pv2 = jnp.where((iv2 >= P) & (iv2 < 2 * P), D, 0)
                offs = [pv1[r] for r in range(LANES)]
                offs += [pv2[r] for r in range(2 * LANES - L, LANES)]
                for c in range(4):
                    acc = rows_ctx[e * L, pl.ds(offs[0] + c * LANES, LANES)]
                    for r in range(1, L):
                        acc = acc + rows_ctx[
                            e * L + r, pl.ds(offs[r] + c * LANES, LANES)]
                    meanb[pl.ds(e * D + c * LANES, LANES)] = acc * rcp
                return carry2

            lax.fori_loop(0, CB, elem, 0)
            pltpu.sync_copy(meanb, out_hbm.at[pl.ds(base * D, CB * D)])
            return carry

        lax.fori_loop(0, NBLK, block, 0)

    return k(ctx_flat, syn0p)


def _sc_dots(tn_flat, mean_flat, syn1p):
    @functools.partial(
        pl.kernel,
        out_type=jax.ShapeDtypeStruct((B * SLOTS,), jnp.float32),
        mesh=plsc.VectorSubcoreMesh(**_MESH),
        compiler_params=_SC_PARAMS,
        scratch_types=[
            pltpu.VMEM((CB2 * TN + LANES,), jnp.int32),
            pltpu.VMEM((CB2 * TN,), jnp.int32),
            pltpu.VMEM((CB2 * TN, PD), jnp.float32),
            pltpu.VMEM((CB2 * D,), jnp.float32),
            pltpu.VMEM((CB2 * SLOTS,), jnp.float32),
            pltpu.SemaphoreType.DMA,
        ],
    )
    def k(tn_hbm, mean_hbm, syn1_hbm, out_hbm,
          idx_tn, idxp_tn, rows_tn, meanb, scores, sem):
        wid = lax.axis_index("s") * NC + lax.axis_index("c")
        lane = lax.iota(jnp.int32, LANES)

        def block(g, carry):
            base = wid * EPW + g * CB2
            hmean = pltpu.async_copy(
                mean_hbm.at[pl.ds(base * D, CB2 * D)], meanb, sem)
            pltpu.sync_copy(tn_hbm.at[pl.ds(base * TN, CB2 * TN)],
                            idx_tn.at[pl.ds(0, CB2 * TN)])
            for t in range(CB2 * TN // LANES):
                v = idx_tn[pl.ds(t * LANES, LANES)]
                idxp_tn[pl.ds(t * LANES, LANES)] = jnp.where(v >= P, v - P, v)
            handles = [hmean]
            for t in range(CB2 * TN // 128):
                handles.append(pltpu.async_copy(
                    syn1_hbm.at[idxp_tn.at[pl.ds(t * 128, 128)]],
                    rows_tn.at[pl.ds(t * 128, 128)], sem))
            for h in handles:
                h.wait()

            def elem(e, carry2):
                mean = [meanb[pl.ds(e * D + c * LANES, LANES)]
                        for c in range(4)]
                tvi = idx_tn[pl.ds(e * TN, LANES)]
                tv = jnp.where((tvi >= P) & (tvi < 2 * P), D, 0)
                s = jnp.full((LANES,), PAD_SCORE, jnp.float32)
                for n in range(TN):
                    off = tv[n]
                    acc = mean[0] * rows_tn[e * TN + n, pl.ds(off, LANES)]
                    for c in range(1, 4):
                        acc = acc + mean[c] * rows_tn[
                            e * TN + n, pl.ds(off + c * LANES, LANES)]
                    val = jnp.sum(acc) if n == 0 else -jnp.sum(acc)
                    s = jnp.where(lane == n, val, s)
                scores[pl.ds(e * SLOTS, SLOTS)] = s
                return carry2

            lax.fori_loop(0, CB2, elem, 0)
            pltpu.sync_copy(scores,
                            out_hbm.at[pl.ds(base * SLOTS, CB2 * SLOTS)])
            return carry

        lax.fori_loop(0, EPW // CB2, block, 0)

    return k(tn_flat, mean_flat, syn1p)


def _tc_loss(scores2d):
    def body(s_ref, o_ref):
        x = s_ref[...]
        ls = jnp.minimum(x, 0.0) - jnp.log1p(jnp.exp(-jnp.abs(x)))
        o_ref[...] = jnp.full((1, 1), -jnp.sum(ls), jnp.float32)

    out = pl.pallas_call(
        body,
        out_shape=jax.ShapeDtypeStruct((1, 1), jnp.float32),
    )(scores2d)
    return out[0, 0]


def kernel(target, context, negatives, syn0, syn1):
    ctx_flat = context.reshape(-1).astype(jnp.int32)
    tn = jnp.concatenate([target[:, None].astype(jnp.int32),
                          negatives.astype(jnp.int32)], axis=1)
    syn0p = _tc_pack_rows(syn0)
    means = _sc_pool(ctx_flat, syn0p)
    syn1p = _tc_pack_rows(syn1)
    scores = _sc_dots(tn.reshape(-1), means, syn1p)
    return _tc_loss(scores.reshape(B * SLOTS // 128, 128))


# speedup vs baseline: 2.0431x; 1.0159x over previous
import functools

import jax
import jax.numpy as jnp
from jax import lax
from jax.experimental import pallas as pl
from jax.experimental.pallas import tpu as pltpu
from jax.experimental.pallas import tpu_sc as plsc

B = 16384
L = 20
NNEG = 5
D = 64
PD = 128
NC, NS, LANES = 2, 16, 16
NW = NC * NS
EPW = B // NW
CB = 32
NBLK = EPW // CB
TN = 1 + NNEG
SLOTS = 16
PAD_SCORE = 1e4
CB2 = 64
TBLK = 16384
P = 491520
NPB = 32
PTB = P // TBLK


def _tc_pack_rows(table):
    t = jnp.transpose(table)

    def body(x1_ref, x2_ref, o_ref):
        eye = (lax.broadcasted_iota(jnp.int32, (D, D), 0)
               == lax.broadcasted_iota(jnp.int32, (D, D), 1)
               ).astype(jnp.float32)
        dn = (((0,), (0,)), ((), ()))
        y1 = lax.dot_general(
            x1_ref[...], eye, dn, preferred_element_type=jnp.float32)
        y2 = lax.dot_general(
            x2_ref[...], eye, dn, preferred_element_type=jnp.float32)
        o_ref[...] = jnp.concatenate([y1, y2], axis=1)

    return pl.pallas_call(
        body,
        grid=(NPB,),
        in_specs=[
            pl.BlockSpec((D, TBLK),
                         lambda j: (0, jnp.where(j >= PTB, j + PTB, j))),
            pl.BlockSpec((D, TBLK), lambda j: (0, PTB + j)),
        ],
        out_specs=pl.BlockSpec((TBLK, PD), lambda j: (j, 0)),
        out_shape=jax.ShapeDtypeStruct((NPB * TBLK, PD), jnp.float32),
        compiler_params=pltpu.CompilerParams(
            dimension_semantics=("arbitrary",),
            fuse_transposed_lhs_in_matmul=True),
    )(t, t)


_SC_PARAMS = pltpu.CompilerParams(
    needs_layout_passes=False, use_tc_tiling_on_sc=True)
_MESH = dict(core_axis_name="c", subcore_axis_name="s")


def _sc_pool(ctx2d, syn0p):
    @functools.partial(
        pl.kernel,
        out_type=jax.ShapeDtypeStruct((B * D,), jnp.float32),
        mesh=plsc.VectorSubcoreMesh(**_MESH),
        compiler_params=_SC_PARAMS,
        scratch_types=[
            pltpu.VMEM((CB, L), jnp.int32),
            pltpu.VMEM((CB * L,), jnp.int32),
            pltpu.VMEM((CB * L, PD), jnp.float32),
            pltpu.VMEM((CB * D,), jnp.float32),
            pltpu.SemaphoreType.DMA,
        ],
    )
    def k(ctx_hbm, syn0_hbm, out_hbm,
          idx_ctx, idxp_ctx, rows_ctx, meanb, sem):
        wid = lax.axis_index("s") * NC + lax.axis_index("c")
        lane = lax.iota(jnp.int32, LANES)

        def block(g, carry):
            base = wid * EPW + g * CB
            pltpu.sync_copy(ctx_hbm.at[pl.ds(base, CB)], idx_ctx)

            def remap(e, carry2):
                w1 = idx_ctx[e, pl.ds(0, LANES)]
                w2 = idx_ctx[e, pl.ds(L - LANES, LANES)]
                idxp_ctx[pl.ds(e * L, LANES)] = jnp.where(w1 >= P, w1 - P, w1)
                idxp_ctx[pl.ds(e * L + (L - LANES), LANES)] = jnp.where(
                    w2 >= P, w2 - P, w2)
                return carry2

            lax.fori_loop(0, CB, remap, 0)
            handles = []
            for t in range(CB * L // 128):
                handles.append(pltpu.async_copy(
                    syn0_hbm.at[idxp_ctx.at[pl.ds(t * 128, 128)]],
                    rows_ctx.at[pl.ds(t * 128, 128)], sem))
            for h in handles:
                h.wait()

            def elem(e, carry2):
                iv1 = idx_ctx[e, pl.ds(0, LANES)]
                iv2 = idx_ctx[e, pl.ds(L - LANES, LANES)]
                cnt = (jnp.sum(jnp.where(iv1 != 0, 1.0, 0.0))
                       + jnp.sum(jnp.where((iv2 != 0) & (lane >= 2 * LANES - L),
                                           1.0, 0.0)))
                rcp = 1.0 / jnp.full((LANES,), cnt, jnp.float32)
                pv1 = jnp.where((iv1 >= P) & (iv1 < 2 * P), D, 0)
                pv2 = jnp.where((iv2 >= P) & (iv2 < 2 * P), D, 0)
                offs = [pv1[r] for r in range(LANES)]
                offs += [pv2[r] for r in range(2 * LANES - L, LANES)]
                for c in range(4):
                    acc = rows_ctx[e * L, pl.ds(offs[0] + c * LANES, LANES)]
                    for r in range(1, L):
                        acc = acc + rows_ctx[
                            e * L + r, pl.ds(offs[r] + c * LANES, LANES)]
                    meanb[pl.ds(e * D + c * LANES, LANES)] = acc * rcp
                return carry2

            lax.fori_loop(0, CB, elem, 0)
            pltpu.sync_copy(meanb, out_hbm.at[pl.ds(base * D, CB * D)])
            return carry

        lax.fori_loop(0, NBLK, block, 0)

    return k(ctx2d, syn0p)


def _sc_dots(tn_flat, mean_flat, syn1p):
    @functools.partial(
        pl.kernel,
        out_type=jax.ShapeDtypeStruct((B * SLOTS,), jnp.float32),
        mesh=plsc.VectorSubcoreMesh(**_MESH),
        compiler_params=_SC_PARAMS,
        scratch_types=[
            pltpu.VMEM((CB2 * TN + LANES,), jnp.int32),
            pltpu.VMEM((CB2 * TN,), jnp.int32),
            pltpu.VMEM((CB2 * TN, PD), jnp.float32),
            pltpu.VMEM((CB2 * D,), jnp.float32),
            pltpu.VMEM((CB2 * SLOTS,), jnp.float32),
            pltpu.SemaphoreType.DMA,
        ],
    )
    def k(tn_hbm, mean_hbm, syn1_hbm, out_hbm,
          idx_tn, idxp_tn, rows_tn, meanb, scores, sem):
        wid = lax.axis_index("s") * NC + lax.axis_index("c")
        lane = lax.iota(jnp.int32, LANES)

        def block(g, carry):
            base = wid * EPW + g * CB2
            hmean = pltpu.async_copy(
                mean_hbm.at[pl.ds(base * D, CB2 * D)], meanb, sem)
            pltpu.sync_copy(tn_hbm.at[pl.ds(base * TN, CB2 * TN)],
                            idx_tn.at[pl.ds(0, CB2 * TN)])
            for t in range(CB2 * TN // LANES):
                v = idx_tn[pl.ds(t * LANES, LANES)]
                idxp_tn[pl.ds(t * LANES, LANES)] = jnp.where(v >= P, v - P, v)
            handles = [hmean]
            for t in range(CB2 * TN // 128):
                handles.append(pltpu.async_copy(
                    syn1_hbm.at[idxp_tn.at[pl.ds(t * 128, 128)]],
                    rows_tn.at[pl.ds(t * 128, 128)], sem))
            for h in handles:
                h.wait()

            def elem(e, carry2):
                mean = [meanb[pl.ds(e * D + c * LANES, LANES)]
                        for c in range(4)]
                tvi = idx_tn[pl.ds(e * TN, LANES)]
                tv = jnp.where((tvi >= P) & (tvi < 2 * P), D, 0)
                s = jnp.full((LANES,), PAD_SCORE, jnp.float32)
                for n in range(TN):
                    off = tv[n]
                    acc = mean[0] * rows_tn[e * TN + n, pl.ds(off, LANES)]
                    for c in range(1, 4):
                        acc = acc + mean[c] * rows_tn[
                            e * TN + n, pl.ds(off + c * LANES, LANES)]
                    val = jnp.sum(acc) if n == 0 else -jnp.sum(acc)
                    s = jnp.where(lane == n, val, s)
                scores[pl.ds(e * SLOTS, SLOTS)] = s
                return carry2

            lax.fori_loop(0, CB2, elem, 0)
            pltpu.sync_copy(scores,
                            out_hbm.at[pl.ds(base * SLOTS, CB2 * SLOTS)])
            return carry

        lax.fori_loop(0, EPW // CB2, block, 0)

    return k(tn_flat, mean_flat, syn1p)


def _tc_loss(scores2d):
    def body(s_ref, o_ref):
        x = s_ref[...]
        ls = jnp.minimum(x, 0.0) - jnp.log1p(jnp.exp(-jnp.abs(x)))
        o_ref[...] = jnp.full((1, 1), -jnp.sum(ls), jnp.float32)

    out = pl.pallas_call(
        body,
        out_shape=jax.ShapeDtypeStruct((1, 1), jnp.float32),
    )(scores2d)
    return out[0, 0]


def kernel(target, context, negatives, syn0, syn1):
    tn = jnp.concatenate([target[:, None].astype(jnp.int32),
                          negatives.astype(jnp.int32)], axis=1)
    syn0p = _tc_pack_rows(syn0)
    means = _sc_pool(context.astype(jnp.int32), syn0p)
    syn1p = _tc_pack_rows(syn1)
    scores = _sc_dots(tn.reshape(-1), means, syn1p)
    return _tc_loss(scores.reshape(B * SLOTS // 128, 128))
